# Initial kernel scaffold; baseline (speedup 1.0000x reference)
import jax
import jax.numpy as jnp
from jax.experimental import pallas as pl


def kernel(x, edge_index, W, b):
    raise NotImplementedError("write your pallas kernel here")


# speedup vs baseline: 15.6576x; 15.6576x over previous
import functools

import jax
import jax.numpy as jnp
from jax import lax
from jax.experimental import pallas as pl
from jax.experimental.pallas import tpu as pltpu
from jax.experimental.pallas import tpu_sc as plsc

N = 10000
D = 128
E = 320000

NC = 2
NS = 16
K = 128

NPAD = 10112
ROWS_PER_TILE = NPAD // NS

CHUNKS = -(-E // (NC * NS * K))
E_PER_TILE = CHUNKS * K
E_PER_CORE = NS * E_PER_TILE
E_PAD = NC * E_PER_CORE

_MESH = plsc.VectorSubcoreMesh(core_axis_name="c", subcore_axis_name="s")


NPD = 16384
CPT = NPD // NS


@functools.partial(
    pl.kernel,
    out_type=jax.ShapeDtypeStruct((NC, 128, 128), jnp.int32),
    mesh=_MESH,
    scratch_types=[
        pltpu.VMEM((E_PER_TILE,), jnp.int32),
        pltpu.VMEM((NPD,), jnp.int32),
        pltpu.VMEM((NS, CPT), jnp.int32),
        pltpu.VMEM((8, 128), jnp.int32),
        pltpu.VMEM_SHARED((NS, NPD), jnp.int32),
    ],
    compiler_params=pltpu.CompilerParams(needs_layout_passes=False),
)
def _deg_kernel(dst_hbm, zeros_hbm, out_hbm, didx, hist, redbuf, res, shared):
    c = lax.axis_index("c")
    s = lax.axis_index("s")
    pltpu.sync_copy(zeros_hbm, hist)
    pltpu.sync_copy(dst_hbm.at[pl.ds(c * E_PER_CORE + s * E_PER_TILE,
                                     E_PER_TILE)], didx)
    ones16 = jnp.full((16,), 1, jnp.int32)

    def body(j, carry):
        idx = didx[pl.ds(j * 16, 16)]
        plsc.addupdate_scatter(hist, [idx], ones16)
        return carry

    lax.fori_loop(0, E_PER_TILE // 16, body, 0)
    pltpu.sync_copy(hist, shared.at[s])
    plsc.subcore_barrier()

    pltpu.sync_copy(shared.at[:, pl.ds(s * CPT, CPT)], redbuf)

    def rbody(j, carry):
        acc16 = redbuf[0, pl.ds(j * 16, 16)]
        for r in range(1, NS):
            acc16 = acc16 + redbuf[r, pl.ds(j * 16, 16)]
        res[j // 8, pl.ds((j % 8) * 16, 16)] = acc16
        return carry

    lax.fori_loop(0, CPT // 16, rbody, 0)
    pltpu.sync_copy(res, out_hbm.at[c, pl.ds(s * 8, 8)])


@functools.partial(
    pl.kernel,
    out_type=jax.ShapeDtypeStruct((NC, NPAD, D), jnp.float32),
    mesh=_MESH,
    scratch_types=[
        pltpu.VMEM((K,), jnp.int32),
        pltpu.VMEM((K,), jnp.int32),
        pltpu.VMEM((K, D), jnp.float32),
        pltpu.VMEM_SHARED((NPAD, D), jnp.float32),
        pltpu.SemaphoreType.DMA,
    ],
)
def _agg_kernel(y_hbm, src_hbm, dst_hbm, zeros_hbm, out_hbm,
                sidx, didx, rows, acc, sem):
    c = lax.axis_index("c")
    s = lax.axis_index("s")
    r0 = s * ROWS_PER_TILE
    pltpu.sync_copy(zeros_hbm.at[pl.ds(r0, ROWS_PER_TILE)],
                    acc.at[pl.ds(r0, ROWS_PER_TILE)])
    plsc.subcore_barrier()

    def body(i, carry):
        base = c * E_PER_CORE + s * E_PER_TILE + i * K
        pltpu.sync_copy(src_hbm.at[pl.ds(base, K)], sidx)
        pltpu.sync_copy(dst_hbm.at[pl.ds(base, K)], didx)
        pltpu.async_copy(y_hbm.at[sidx], rows, sem).wait()
        pltpu.sync_copy(rows, acc.at[didx], add=True)
        return carry

    lax.fori_loop(0, CHUNKS, body, 0)
    plsc.subcore_barrier()
    pltpu.sync_copy(acc.at[pl.ds(r0, ROWS_PER_TILE)],
                    out_hbm.at[c, pl.ds(r0, ROWS_PER_TILE)])


def _y_body(deg0_ref, deg1_ref, x_ref, w_ref, y_ref):
    cnt = (deg0_ref[...] + deg1_ref[...]).astype(jnp.float32)
    dis = lax.rsqrt(cnt + 1.0)
    xw = jnp.dot(x_ref[...], w_ref[...], preferred_element_type=jnp.float32)
    y_ref[...] = xw * dis


def _out_body(deg0_ref, deg1_ref, acc0_ref, acc1_ref, y_ref, b_ref, o_ref):
    cnt = (deg0_ref[...] + deg1_ref[...]).astype(jnp.float32)
    dis = lax.rsqrt(cnt + 1.0)
    o_ref[...] = dis * (acc0_ref[...] + acc1_ref[...] + y_ref[...]) + b_ref[...]


_BLK = 1000
_GRID = N // _BLK


def kernel(x, edge_index, W, b):
    src = edge_index[0].astype(jnp.int32)
    dst = edge_index[1].astype(jnp.int32)
    pad = E_PAD - E
    src_p = jnp.concatenate([src, jnp.zeros((pad,), jnp.int32)])
    dst_p = jnp.concatenate([dst, jnp.full((pad,), N, jnp.int32)])

    zeros_h = jnp.zeros((NPD,), jnp.int32)
    zeros_d = jnp.zeros((NPAD, D), jnp.float32)

    deg = _deg_kernel(dst_p, zeros_h)
    deg0 = deg[0].reshape(NPD)[:N].reshape(N, 1)
    deg1 = deg[1].reshape(NPD)[:N].reshape(N, 1)

    y = pl.pallas_call(
        _y_body,
        grid=(_GRID,),
        in_specs=[
            pl.BlockSpec((_BLK, 1), lambda i: (i, 0)),
            pl.BlockSpec((_BLK, 1), lambda i: (i, 0)),
            pl.BlockSpec((_BLK, D), lambda i: (i, 0)),
            pl.BlockSpec((D, D), lambda i: (0, 0)),
        ],
        out_specs=pl.BlockSpec((_BLK, D), lambda i: (i, 0)),
        out_shape=jax.ShapeDtypeStruct((N, D), jnp.float32),
    )(deg0, deg1, x, W)

    acc = _agg_kernel(y, src_p, dst_p, zeros_d)
    acc0 = acc[0, :N]
    acc1 = acc[1, :N]

    out = pl.pallas_call(
        _out_body,
        grid=(_GRID,),
        in_specs=[
            pl.BlockSpec((_BLK, 1), lambda i: (i, 0)),
            pl.BlockSpec((_BLK, 1), lambda i: (i, 0)),
            pl.BlockSpec((_BLK, D), lambda i: (i, 0)),
            pl.BlockSpec((_BLK, D), lambda i: (i, 0)),
            pl.BlockSpec((_BLK, D), lambda i: (i, 0)),
            pl.BlockSpec((1, D), lambda i: (0, 0)),
        ],
        out_specs=pl.BlockSpec((_BLK, D), lambda i: (i, 0)),
        out_shape=jax.ShapeDtypeStruct((N, D), jnp.float32),
    )(deg0, deg1, acc0, acc1, y, b.reshape(1, D))

    return out
